# baseline (speedup 1.0000x reference)
import functools

import jax
import jax.numpy as jnp
import numpy as np
from jax import lax
from jax.experimental import pallas as pl
from jax.experimental.pallas import tpu as pltpu
from jax.experimental.pallas import tpu_sc as plsc

N_ATOMS = 50000
N_EDGES = 800000
DIM = 3
NUM_CHAINS = 4
HIDDEN = 32
KB = 8.617333262e-05
T_IN_K = 300.0
TTIME = 20.0
KT = T_IN_K * KB
N_DOF = N_ATOMS * DIM
TARGET_KE = 0.5 * N_DOF * KT
QB = 2.0 * np.array([N_DOF * KT * TTIME ** 2] + [KT * TTIME ** 2] * (NUM_CHAINS - 1),
                    dtype=np.float32)

NC = 2
NS = 16
NW = NC * NS
LANES = 16

CH = 128
EPAD = 819200
EW = EPAD // NW
NIT = EW // CH
NP = 50048
HOLE0 = 24992
HOLEN = 128
BOT = 64
NPH = NP + HOLEN + BOT
RPT = NPH // NS


def _rsqrt16(s):
    i = plsc.bitcast(s, jnp.int32)
    i = jnp.full((16,), 0x5F3759DF, jnp.int32) - (i >> 1)
    y = plsc.bitcast(i, jnp.float32)
    hs = s * 0.5
    for _ in range(3):
        y = y * (1.5 - hs * y * y)
    return y


NB = CH // 32


def _sc_body(q4_hbm, src2_hbm, dst2_hbm, src2s_hbm, dst2s_hbm,
             w1_hbm, b1_hbm, c4_hbm, z_hbm, out_hbm, refs, gsem):
    ibs = refs[0:4]
    ibd = refs[4:8]
    sbs = refs[8:12]
    sbd = refs[12:16]
    qs, qd, cpos, cneg, w1v, b1v, c4v, zb, fsh = refs[16:]
    cid = lax.axis_index("c")
    sid = lax.axis_index("s")
    wid = cid * NS + sid

    pltpu.sync_copy(w1_hbm, w1v)
    pltpu.sync_copy(b1_hbm, b1v)
    pltpu.sync_copy(c4_hbm, c4v)

    r0 = sid * RPT
    pltpu.sync_copy(z_hbm.at[pl.ds(r0, RPT)], zb)
    pltpu.sync_copy(zb, fsh.at[pl.ds(r0, RPT)])

    zero16 = jnp.zeros((16,), jnp.float32)
    for b in range(NB):
        for g in range(2, 8):
            lanes = lax.iota(jnp.int32, 16) + (128 * b + 16 * g)
            for c in range(4):
                colv = jnp.full((16,), c, jnp.int32)
                plsc.store_scatter(cpos, [lanes, colv], zero16)
                plsc.store_scatter(cneg, [lanes, colv], zero16)
        for g in range(2):
            lanes = lax.iota(jnp.int32, 16) + (128 * b + 16 * g)
            col3 = jnp.full((16,), 3, jnp.int32)
            plsc.store_scatter(cpos, [lanes, col3], zero16)
            plsc.store_scatter(cneg, [lanes, col3], zero16)

    for b in range(NB):
        lanes2 = lax.iota(jnp.int32, 16) * 2
        zi16 = jnp.zeros((16,), jnp.int32)
        for g in range(2):
            pos = lanes2 + (64 + 32 * g)
            val = (lax.iota(jnp.int32, 16) + 16 * g) * 2
            plsc.store_scatter(sbs[b], [pos], val)
            plsc.store_scatter(sbs[b], [pos + 1], zi16)
            plsc.store_scatter(sbd[b], [pos], val)
            plsc.store_scatter(sbd[b], [pos + 1], zi16)

    plsc.subcore_barrier()

    def chunk(it, carry):
        base = wid * EW + it * CH
        for b in range(NB):
            off = 2 * (base + 32 * b)
            pltpu.sync_copy(src2_hbm.at[pl.ds(off, 64)], ibs[b])
            pltpu.sync_copy(dst2_hbm.at[pl.ds(off, 64)], ibd[b])
            pltpu.sync_copy(src2s_hbm.at[pl.ds(off, 64)],
                            sbs[b].at[pl.ds(0, 64)])
            pltpu.sync_copy(dst2s_hbm.at[pl.ds(off, 64)],
                            sbd[b].at[pl.ds(0, 64)])
        cps = []
        for b in range(NB):
            cps.append(pltpu.async_copy(q4_hbm.at[ibs[b]],
                                        qs.at[pl.ds(64 * b, 64)], gsem))
            cps.append(pltpu.async_copy(q4_hbm.at[ibd[b]],
                                        qd.at[pl.ds(64 * b, 64)], gsem))
        for cp in cps:
            cp.wait()

        col0 = jnp.zeros((16,), jnp.int32)
        col1 = jnp.full((16,), 1, jnp.int32)
        col2 = jnp.full((16,), 2, jnp.int32)
        wv = [w1v[pl.ds(0, 16)], w1v[pl.ds(16, 16)]]
        bv = [b1v[pl.ds(0, 16)], b1v[pl.ds(16, 16)]]
        cv = [c4v[pl.ds(0, 16)], c4v[pl.ds(16, 16)]]
        ws = [wv[h // 16][h % 16] for h in range(HIDDEN)]
        bs = [bv[h // 16][h % 16] for h in range(HIDDEN)]
        cs = [cv[h // 16][h % 16] for h in range(HIDDEN)]
        for g in range(CH // LANES):
            lanes = lax.iota(jnp.int32, 16) + (64 * (g // 2) + 16 * (g % 2))
            sx = plsc.load_gather(qs, [lanes, col0])
            sy = plsc.load_gather(qs, [lanes, col1])
            sz = plsc.load_gather(qs, [lanes, col2])
            dx = plsc.load_gather(qd, [lanes, col0])
            dy = plsc.load_gather(qd, [lanes, col1])
            dz = plsc.load_gather(qd, [lanes, col2])
            rx = dx - sx
            ry = dy - sy
            rz = dz - sz
            s2 = rx * rx + ry * ry + rz * rz + 1e-9
            irt = _rsqrt16(s2)
            d = s2 * irt
            gacc = jnp.zeros((16,), jnp.float32)
            for h in range(HIDDEN):
                x = d * ws[h] + bs[h]
                u = jnp.exp(jnp.abs(x) * -2.0)
                dn = 1.0 + u
                gacc = gacc + (cs[h] * u) / (dn * dn)
            scale = gacc * irt
            cx = scale * rx
            cy = scale * ry
            cz = scale * rz
            clanes = lax.iota(jnp.int32, 16) + (128 * (g // 2) + 16 * (g % 2))
            plsc.store_scatter(cpos, [clanes, col0], cx)
            plsc.store_scatter(cpos, [clanes, col1], cy)
            plsc.store_scatter(cpos, [clanes, col2], cz)
            plsc.store_scatter(cneg, [clanes, col0], -cx)
            plsc.store_scatter(cneg, [clanes, col1], -cy)
            plsc.store_scatter(cneg, [clanes, col2], -cz)

        for b in range(NB):
            pltpu.sync_copy(cpos.at[pl.ds(128 * b, 128)], fsh.at[sbd[b]],
                            add=True)
            pltpu.sync_copy(cneg.at[pl.ds(128 * b, 128)], fsh.at[sbs[b]],
                            add=True)
        return carry

    lax.fori_loop(0, NIT, chunk, 0)
    plsc.subcore_barrier()

    pltpu.sync_copy(fsh.at[pl.ds(r0, RPT)], zb)
    pltpu.sync_copy(zb, out_hbm.at[cid, pl.ds(r0, RPT)])


def _sc_entry(q4_hbm, src2_hbm, dst2_hbm, src2s_hbm, dst2s_hbm,
              w1_hbm, b1_hbm, c4_hbm, z_hbm, out_hbm, *refs):
    _sc_body(q4_hbm, src2_hbm, dst2_hbm, src2s_hbm, dst2s_hbm,
             w1_hbm, b1_hbm, c4_hbm, z_hbm, out_hbm, refs[:-1], refs[-1])


_sc_kernel = functools.partial(
    pl.kernel,
    out_type=jax.ShapeDtypeStruct((NC, NPH, 4), jnp.float32),
    mesh=plsc.VectorSubcoreMesh(core_axis_name="c", subcore_axis_name="s"),
    compiler_params=pltpu.CompilerParams(needs_layout_passes=False,
                                         use_tc_tiling_on_sc=False),
    scratch_types=(
        [pltpu.VMEM((64,), jnp.int32) for _ in range(8)] +
        [pltpu.VMEM((128,), jnp.int32) for _ in range(8)] + [
            pltpu.VMEM((64 * NB, 4), jnp.float32),
            pltpu.VMEM((64 * NB, 4), jnp.float32),
            pltpu.VMEM((128 * NB, 4), jnp.float32),
            pltpu.VMEM((128 * NB, 4), jnp.float32),
            pltpu.VMEM((HIDDEN,), jnp.float32),
            pltpu.VMEM((HIDDEN,), jnp.float32),
            pltpu.VMEM((HIDDEN,), jnp.float32),
            pltpu.VMEM((RPT, 4), jnp.float32),
            pltpu.VMEM_SHARED((NPH, 4), jnp.float32),
            pltpu.SemaphoreType.DMA,
        ]),
)(_sc_entry)


def _tc_body(parts_ref, v_ref, m_ref, pe_ref, dvdt_ref, dpeta_ref):
    gsum = parts_ref[0] + parts_ref[1]
    p0 = pe_ref[0]
    p1 = pe_ref[1]
    p2 = pe_ref[2]
    p3 = pe_ref[3]
    vv = v_ref[...]
    mm = m_ref[...]
    dvdt_ref[...] = -gsum / mm - (p0 / QB[0]) * vv
    ke = 0.5 * jnp.sum(mm * vv * vv)
    dpeta_ref[0] = 2.0 * (ke - TARGET_KE) - p0 * p1 / QB[1]
    dpeta_ref[1] = p0 * p0 / QB[0] - KT - p1 * p2 / QB[2]
    dpeta_ref[2] = p1 * p1 / QB[1] - KT - p2 * p3 / QB[3]
    dpeta_ref[3] = p2 * p2 / QB[2] - KT


_FLAT = (NP * 4) // 128


def kernel(v, q, p_eta, nbr_list, mass, W1, b1, W2, b2):
    q4 = jnp.concatenate(
        [q, jnp.zeros((N_ATOMS, 1), jnp.float32)], axis=1)
    q4 = jnp.pad(q4, ((0, NP - N_ATOMS), (0, 0)))
    pad_ids = (jnp.arange(N_EDGES, EPAD, dtype=jnp.int32) % N_ATOMS)
    srcp = jnp.concatenate([nbr_list[0], pad_ids])
    dstp = jnp.concatenate([nbr_list[1], pad_ids])
    srcs = srcp + jnp.where(srcp >= HOLE0, HOLEN + BOT, BOT).astype(jnp.int32)
    dsts = dstp + jnp.where(dstp >= HOLE0, HOLEN + BOT, BOT).astype(jnp.int32)
    zz = jnp.zeros_like(srcp)
    src2 = jnp.stack([2 * srcp, zz], axis=1).reshape(-1)
    dst2 = jnp.stack([2 * dstp, zz], axis=1).reshape(-1)
    src2s = jnp.stack([2 * srcs, zz], axis=1).reshape(-1)
    dst2s = jnp.stack([2 * dsts, zz], axis=1).reshape(-1)
    w1 = W1[0]
    c4 = 4.0 * W1[0] * W2[:, 0]
    zrows = jnp.zeros((NPH, 4), jnp.float32)

    partsh = _sc_kernel(q4, src2, dst2, src2s, dst2s, w1, b1, c4, zrows)
    parts = jnp.concatenate(
        [partsh[:, BOT:BOT + HOLE0],
         partsh[:, BOT + HOLE0 + HOLEN:BOT + HOLE0 + HOLEN + (NP - HOLE0)]],
        axis=1)

    v4 = jnp.pad(v, ((0, NP - N_ATOMS), (0, 1))).reshape(_FLAT, 128)
    m4 = jnp.broadcast_to(
        jnp.pad(mass, (0, NP - N_ATOMS), constant_values=1.0)[:, None],
        (NP, 4)).reshape(_FLAT, 128)
    parts_flat = parts.reshape(NC, _FLAT, 128)

    dvdt_flat, dpeta = pl.pallas_call(
        _tc_body,
        out_shape=[
            jax.ShapeDtypeStruct((_FLAT, 128), jnp.float32),
            jax.ShapeDtypeStruct((NUM_CHAINS,), jnp.float32),
        ],
        in_specs=[
            pl.BlockSpec(memory_space=pltpu.VMEM),
            pl.BlockSpec(memory_space=pltpu.VMEM),
            pl.BlockSpec(memory_space=pltpu.VMEM),
            pl.BlockSpec(memory_space=pltpu.SMEM),
        ],
        out_specs=[
            pl.BlockSpec(memory_space=pltpu.VMEM),
            pl.BlockSpec(memory_space=pltpu.SMEM),
        ],
    )(parts_flat, v4, m4, p_eta)

    dvdt = dvdt_flat.reshape(NP, 4)[:N_ATOMS, :DIM]
    return (dvdt, v, dpeta)


# speedup vs baseline: 1.0014x; 1.0014x over previous
import functools

import jax
import jax.numpy as jnp
import numpy as np
from jax import lax
from jax.experimental import pallas as pl
from jax.experimental.pallas import tpu as pltpu
from jax.experimental.pallas import tpu_sc as plsc

N_ATOMS = 50000
N_EDGES = 800000
DIM = 3
NUM_CHAINS = 4
HIDDEN = 32
KB = 8.617333262e-05
T_IN_K = 300.0
TTIME = 20.0
KT = T_IN_K * KB
N_DOF = N_ATOMS * DIM
TARGET_KE = 0.5 * N_DOF * KT
QB = 2.0 * np.array([N_DOF * KT * TTIME ** 2] + [KT * TTIME ** 2] * (NUM_CHAINS - 1),
                    dtype=np.float32)

NC = 2
NS = 16
NW = NC * NS
LANES = 16

CH = 128
EPAD = 819200
EW = EPAD // NW
NIT = EW // CH
NP = 50048
HOLE0 = 24992
HOLEN = 128
BOT = 64
NPH = NP + HOLEN + BOT
RPT = NPH // NS


def _rsqrt16(s):
    i = plsc.bitcast(s, jnp.int32)
    i = jnp.full((16,), 0x5F3759DF, jnp.int32) - (i >> 1)
    y = plsc.bitcast(i, jnp.float32)
    hs = s * 0.5
    for _ in range(3):
        y = y * (1.5 - hs * y * y)
    return y


NB = CH // 32


def _sc_body(q4_hbm, src2_hbm, dst2_hbm, src2s_hbm, dst2s_hbm,
             w1_hbm, b1_hbm, c4_hbm, z_hbm, out_hbm, refs, gsem):
    ibs = refs[0:4]
    ibd = refs[4:8]
    sbs = refs[8:12]
    sbd = refs[12:16]
    qs, qd, cpos, cneg, w1v, b1v, c4v, zb, fsh = refs[16:]
    cid = lax.axis_index("c")
    sid = lax.axis_index("s")
    wid = cid * NS + sid

    pltpu.sync_copy(w1_hbm, w1v)
    pltpu.sync_copy(b1_hbm, b1v)
    pltpu.sync_copy(c4_hbm, c4v)

    r0 = sid * RPT
    pltpu.sync_copy(z_hbm.at[pl.ds(r0, RPT)], zb)
    pltpu.sync_copy(zb, fsh.at[pl.ds(r0, RPT)])

    zero16 = jnp.zeros((16,), jnp.float32)
    for b in range(NB):
        for g in range(2, 8):
            lanes = lax.iota(jnp.int32, 16) + (128 * b + 16 * g)
            for c in range(4):
                colv = jnp.full((16,), c, jnp.int32)
                plsc.store_scatter(cpos, [lanes, colv], zero16)
                plsc.store_scatter(cneg, [lanes, colv], zero16)
        for g in range(2):
            lanes = lax.iota(jnp.int32, 16) + (128 * b + 16 * g)
            col3 = jnp.full((16,), 3, jnp.int32)
            plsc.store_scatter(cpos, [lanes, col3], zero16)
            plsc.store_scatter(cneg, [lanes, col3], zero16)

    for b in range(NB):
        lanes2 = lax.iota(jnp.int32, 16) * 2
        zi16 = jnp.zeros((16,), jnp.int32)
        for g in range(2):
            pos = lanes2 + (64 + 32 * g)
            val = (lax.iota(jnp.int32, 16) + 16 * g) * 2
            plsc.store_scatter(sbs[b], [pos], val)
            plsc.store_scatter(sbs[b], [pos + 1], zi16)
            plsc.store_scatter(sbd[b], [pos], val)
            plsc.store_scatter(sbd[b], [pos + 1], zi16)

    plsc.subcore_barrier()

    def chunk(it, carry):
        base = wid * EW + it * CH
        scps = []
        for b in range(NB):
            off = 2 * (base + 32 * b)
            scps.append(pltpu.async_copy(src2_hbm.at[pl.ds(off, 64)],
                                         ibs[b], gsem))
            scps.append(pltpu.async_copy(dst2_hbm.at[pl.ds(off, 64)],
                                         ibd[b], gsem))
            scps.append(pltpu.async_copy(src2s_hbm.at[pl.ds(off, 64)],
                                         sbs[b].at[pl.ds(0, 64)], gsem))
            scps.append(pltpu.async_copy(dst2s_hbm.at[pl.ds(off, 64)],
                                         sbd[b].at[pl.ds(0, 64)], gsem))
        for cp in scps:
            cp.wait()
        cps = []
        for b in range(NB):
            cps.append(pltpu.async_copy(q4_hbm.at[ibs[b]],
                                        qs.at[pl.ds(64 * b, 64)], gsem))
            cps.append(pltpu.async_copy(q4_hbm.at[ibd[b]],
                                        qd.at[pl.ds(64 * b, 64)], gsem))
        for cp in cps:
            cp.wait()

        col0 = jnp.zeros((16,), jnp.int32)
        col1 = jnp.full((16,), 1, jnp.int32)
        col2 = jnp.full((16,), 2, jnp.int32)
        wv = [w1v[pl.ds(0, 16)], w1v[pl.ds(16, 16)]]
        bv = [b1v[pl.ds(0, 16)], b1v[pl.ds(16, 16)]]
        cv = [c4v[pl.ds(0, 16)], c4v[pl.ds(16, 16)]]
        ws = [wv[h // 16][h % 16] for h in range(HIDDEN)]
        bs = [bv[h // 16][h % 16] for h in range(HIDDEN)]
        cs = [cv[h // 16][h % 16] for h in range(HIDDEN)]
        for g in range(CH // LANES):
            lanes = lax.iota(jnp.int32, 16) + (64 * (g // 2) + 16 * (g % 2))
            sx = plsc.load_gather(qs, [lanes, col0])
            sy = plsc.load_gather(qs, [lanes, col1])
            sz = plsc.load_gather(qs, [lanes, col2])
            dx = plsc.load_gather(qd, [lanes, col0])
            dy = plsc.load_gather(qd, [lanes, col1])
            dz = plsc.load_gather(qd, [lanes, col2])
            rx = dx - sx
            ry = dy - sy
            rz = dz - sz
            s2 = rx * rx + ry * ry + rz * rz + 1e-9
            irt = _rsqrt16(s2)
            d = s2 * irt
            gacc = jnp.zeros((16,), jnp.float32)
            for h in range(HIDDEN):
                x = d * ws[h] + bs[h]
                u = jnp.exp(jnp.abs(x) * -2.0)
                dn = 1.0 + u
                gacc = gacc + (cs[h] * u) / (dn * dn)
            scale = gacc * irt
            cx = scale * rx
            cy = scale * ry
            cz = scale * rz
            clanes = lax.iota(jnp.int32, 16) + (128 * (g // 2) + 16 * (g % 2))
            plsc.store_scatter(cpos, [clanes, col0], cx)
            plsc.store_scatter(cpos, [clanes, col1], cy)
            plsc.store_scatter(cpos, [clanes, col2], cz)
            plsc.store_scatter(cneg, [clanes, col0], -cx)
            plsc.store_scatter(cneg, [clanes, col1], -cy)
            plsc.store_scatter(cneg, [clanes, col2], -cz)

        wcps = []
        for b in range(NB):
            wcps.append(pltpu.async_copy(cpos.at[pl.ds(128 * b, 128)],
                                         fsh.at[sbd[b]], gsem, add=True))
            wcps.append(pltpu.async_copy(cneg.at[pl.ds(128 * b, 128)],
                                         fsh.at[sbs[b]], gsem, add=True))
        for cp in wcps:
            cp.wait()
        return carry

    lax.fori_loop(0, NIT, chunk, 0)
    plsc.subcore_barrier()

    pltpu.sync_copy(fsh.at[pl.ds(r0, RPT)], zb)
    pltpu.sync_copy(zb, out_hbm.at[cid, pl.ds(r0, RPT)])


def _sc_entry(q4_hbm, src2_hbm, dst2_hbm, src2s_hbm, dst2s_hbm,
              w1_hbm, b1_hbm, c4_hbm, z_hbm, out_hbm, *refs):
    _sc_body(q4_hbm, src2_hbm, dst2_hbm, src2s_hbm, dst2s_hbm,
             w1_hbm, b1_hbm, c4_hbm, z_hbm, out_hbm, refs[:-1], refs[-1])


_sc_kernel = functools.partial(
    pl.kernel,
    out_type=jax.ShapeDtypeStruct((NC, NPH, 4), jnp.float32),
    mesh=plsc.VectorSubcoreMesh(core_axis_name="c", subcore_axis_name="s"),
    compiler_params=pltpu.CompilerParams(needs_layout_passes=False,
                                         use_tc_tiling_on_sc=False),
    scratch_types=(
        [pltpu.VMEM((64,), jnp.int32) for _ in range(8)] +
        [pltpu.VMEM((128,), jnp.int32) for _ in range(8)] + [
            pltpu.VMEM((64 * NB, 4), jnp.float32),
            pltpu.VMEM((64 * NB, 4), jnp.float32),
            pltpu.VMEM((128 * NB, 4), jnp.float32),
            pltpu.VMEM((128 * NB, 4), jnp.float32),
            pltpu.VMEM((HIDDEN,), jnp.float32),
            pltpu.VMEM((HIDDEN,), jnp.float32),
            pltpu.VMEM((HIDDEN,), jnp.float32),
            pltpu.VMEM((RPT, 4), jnp.float32),
            pltpu.VMEM_SHARED((NPH, 4), jnp.float32),
            pltpu.SemaphoreType.DMA,
        ]),
)(_sc_entry)


def _tc_body(parts_ref, v_ref, m_ref, pe_ref, dvdt_ref, dpeta_ref):
    gsum = parts_ref[0] + parts_ref[1]
    p0 = pe_ref[0]
    p1 = pe_ref[1]
    p2 = pe_ref[2]
    p3 = pe_ref[3]
    vv = v_ref[...]
    mm = m_ref[...]
    dvdt_ref[...] = -gsum / mm - (p0 / QB[0]) * vv
    ke = 0.5 * jnp.sum(mm * vv * vv)
    dpeta_ref[0] = 2.0 * (ke - TARGET_KE) - p0 * p1 / QB[1]
    dpeta_ref[1] = p0 * p0 / QB[0] - KT - p1 * p2 / QB[2]
    dpeta_ref[2] = p1 * p1 / QB[1] - KT - p2 * p3 / QB[3]
    dpeta_ref[3] = p2 * p2 / QB[2] - KT


_FLAT = (NP * 4) // 128


def kernel(v, q, p_eta, nbr_list, mass, W1, b1, W2, b2):
    q4 = jnp.concatenate(
        [q, jnp.zeros((N_ATOMS, 1), jnp.float32)], axis=1)
    q4 = jnp.pad(q4, ((0, NP - N_ATOMS), (0, 0)))
    pad_ids = (jnp.arange(N_EDGES, EPAD, dtype=jnp.int32) % N_ATOMS)
    srcp = jnp.concatenate([nbr_list[0], pad_ids])
    dstp = jnp.concatenate([nbr_list[1], pad_ids])
    srcs = srcp + jnp.where(srcp >= HOLE0, HOLEN + BOT, BOT).astype(jnp.int32)
    dsts = dstp + jnp.where(dstp >= HOLE0, HOLEN + BOT, BOT).astype(jnp.int32)
    zz = jnp.zeros_like(srcp)
    src2 = jnp.stack([2 * srcp, zz], axis=1).reshape(-1)
    dst2 = jnp.stack([2 * dstp, zz], axis=1).reshape(-1)
    src2s = jnp.stack([2 * srcs, zz], axis=1).reshape(-1)
    dst2s = jnp.stack([2 * dsts, zz], axis=1).reshape(-1)
    w1 = W1[0]
    c4 = 4.0 * W1[0] * W2[:, 0]
    zrows = jnp.zeros((NPH, 4), jnp.float32)

    partsh = _sc_kernel(q4, src2, dst2, src2s, dst2s, w1, b1, c4, zrows)
    parts = jnp.concatenate(
        [partsh[:, BOT:BOT + HOLE0],
         partsh[:, BOT + HOLE0 + HOLEN:BOT + HOLE0 + HOLEN + (NP - HOLE0)]],
        axis=1)

    v4 = jnp.pad(v, ((0, NP - N_ATOMS), (0, 1))).reshape(_FLAT, 128)
    m4 = jnp.broadcast_to(
        jnp.pad(mass, (0, NP - N_ATOMS), constant_values=1.0)[:, None],
        (NP, 4)).reshape(_FLAT, 128)
    parts_flat = parts.reshape(NC, _FLAT, 128)

    dvdt_flat, dpeta = pl.pallas_call(
        _tc_body,
        out_shape=[
            jax.ShapeDtypeStruct((_FLAT, 128), jnp.float32),
            jax.ShapeDtypeStruct((NUM_CHAINS,), jnp.float32),
        ],
        in_specs=[
            pl.BlockSpec(memory_space=pltpu.VMEM),
            pl.BlockSpec(memory_space=pltpu.VMEM),
            pl.BlockSpec(memory_space=pltpu.VMEM),
            pl.BlockSpec(memory_space=pltpu.SMEM),
        ],
        out_specs=[
            pl.BlockSpec(memory_space=pltpu.VMEM),
            pl.BlockSpec(memory_space=pltpu.SMEM),
        ],
    )(parts_flat, v4, m4, p_eta)

    dvdt = dvdt_flat.reshape(NP, 4)[:N_ATOMS, :DIM]
    return (dvdt, v, dpeta)
